# baseline (speedup 1.0000x reference)
import jax
import jax.numpy as jnp
from jax.experimental import pallas as pl


_N_STREAMS = 4


def _router_block(*refs):
    x_refs = refs[:_N_STREAMS]
    wt_ref, b_ref, o_ref = refs[_N_STREAMS:]
    chunk = x_refs[0].shape[0]
    bias = b_ref[...]
    for q, x_ref in enumerate(x_refs):
        logits = jnp.dot(x_ref[...], wt_ref[...],
                         preferred_element_type=jnp.float32) + bias
        m = jnp.max(logits, axis=-1, keepdims=True)
        e = jnp.exp(logits - m)
        o_ref[pl.ds(q * chunk, chunk), :] = (
            e / jnp.sum(e, axis=-1, keepdims=True))


def kernel(x, W, b):
    n_tokens, d_model = x.shape
    n_experts = W.shape[0]
    block_t = 1024
    chunk = block_t // _N_STREAMS
    wt = W.T
    b2 = b.reshape(1, n_experts)
    in_specs = [
        pl.BlockSpec((chunk, d_model),
                     lambda i, q=q: (_N_STREAMS * i + q, 0))
        for q in range(_N_STREAMS)
    ]
    in_specs += [
        pl.BlockSpec((d_model, n_experts), lambda i: (0, 0)),
        pl.BlockSpec((1, n_experts), lambda i: (0, 0)),
    ]
    return pl.pallas_call(
        _router_block,
        grid=(n_tokens // block_t,),
        in_specs=in_specs,
        out_specs=pl.BlockSpec((block_t, n_experts), lambda i: (i, 0)),
        out_shape=jax.ShapeDtypeStruct((n_tokens, n_experts), jnp.float32),
    )(*([x] * _N_STREAMS), wt, b2)


# speedup vs baseline: 1.0053x; 1.0053x over previous
import jax
import jax.numpy as jnp
from jax.experimental import pallas as pl


_N_STREAMS = 2


def _router_block(*refs):
    x_refs = refs[:_N_STREAMS]
    wt_ref, b_ref, o_ref = refs[_N_STREAMS:]
    chunk = x_refs[0].shape[0]
    bias = b_ref[...]
    for q, x_ref in enumerate(x_refs):
        logits = jnp.dot(x_ref[...], wt_ref[...],
                         preferred_element_type=jnp.float32) + bias
        m = jnp.max(logits, axis=-1, keepdims=True)
        e = jnp.exp(logits - m)
        o_ref[pl.ds(q * chunk, chunk), :] = (
            e / jnp.sum(e, axis=-1, keepdims=True))


def kernel(x, W, b):
    n_tokens, d_model = x.shape
    n_experts = W.shape[0]
    block_t = 1024
    chunk = block_t // _N_STREAMS
    wt = W.T
    b2 = b.reshape(1, n_experts)
    in_specs = [
        pl.BlockSpec((chunk, d_model),
                     lambda i, q=q: (_N_STREAMS * i + q, 0))
        for q in range(_N_STREAMS)
    ]
    in_specs += [
        pl.BlockSpec((d_model, n_experts), lambda i: (0, 0)),
        pl.BlockSpec((1, n_experts), lambda i: (0, 0)),
    ]
    return pl.pallas_call(
        _router_block,
        grid=(n_tokens // block_t,),
        in_specs=in_specs,
        out_specs=pl.BlockSpec((block_t, n_experts), lambda i: (i, 0)),
        out_shape=jax.ShapeDtypeStruct((n_tokens, n_experts), jnp.float32),
    )(*([x] * _N_STREAMS), wt, b2)
